# baseline (speedup 1.0000x reference)
import jax
import jax.numpy as jnp
from jax.experimental import pallas as pl

T = 2048
H = 1024
E = 8
FFN = 512
SFFN = 2048
NGROUP = 4
TOPKG = 2
TOPK = 2
NCOLS = E * FFN + SFFN
BC = 512
NJ = NCOLS // BC
NEG = -1e30


def _router_kernel(scores_ref, bias_ref, w_ref):
    scores = scores_ref[...]
    sfc = scores + bias_ref[...]

    eidx = jax.lax.broadcasted_iota(jnp.int32, (T, E), 1)
    gidx8 = eidx // 2
    gsum_full = jnp.zeros((T, E), jnp.float32)
    for g in range(NGROUP):
        in_g = gidx8 == g
        gsum_g = jnp.sum(jnp.where(in_g, sfc, 0.0), axis=-1, keepdims=True)
        gsum_full = jnp.where(in_g, gsum_g, gsum_full)

    m1 = jnp.max(gsum_full, axis=-1, keepdims=True)
    i1 = jnp.min(jnp.where(gsum_full == m1, gidx8, NGROUP),
                 axis=-1, keepdims=True)
    gs2 = jnp.where(gidx8 == i1, NEG, gsum_full)
    m2 = jnp.max(gs2, axis=-1, keepdims=True)
    i2 = jnp.min(jnp.where(gs2 == m2, gidx8, NGROUP),
                 axis=-1, keepdims=True)
    gsel = (gidx8 == i1) | (gidx8 == i2)
    masked = jnp.where(gsel, sfc, NEG)
    e_m1 = jnp.max(masked, axis=-1, keepdims=True)
    e1 = jnp.min(jnp.where(masked == e_m1, eidx, E), axis=-1, keepdims=True)
    masked2 = jnp.where(eidx == e1, NEG, masked)
    e_m2 = jnp.max(masked2, axis=-1, keepdims=True)
    e2 = jnp.min(jnp.where(masked2 == e_m2, eidx, E), axis=-1, keepdims=True)

    sel1 = (eidx == e1)
    sel2 = (eidx == e2)
    w1 = jnp.sum(jnp.where(sel1, scores, 0.0), axis=-1, keepdims=True)
    w2 = jnp.sum(jnp.where(sel2, scores, 0.0), axis=-1, keepdims=True)
    denom = w1 + w2 + 1e-20
    w_ref[...] = (jnp.where(sel1, w1, 0.0) + jnp.where(sel2, w2, 0.0)) / denom


def _ffn_kernel(x_ref, w_ref, wg_ref, wu_ref, wd_ref, out_ref):
    x = x_ref[...]
    wg = wg_ref[...]
    wu = wu_ref[...]
    wd = wd_ref[0]
    g = jax.lax.dot_general(
        x, wg, (((1,), (1,)), ((), ())), preferred_element_type=jnp.float32)
    u = jax.lax.dot_general(
        x, wu, (((1,), (1,)), ((), ())), preferred_element_type=jnp.float32)
    a = g * jax.nn.sigmoid(g) * u * w_ref[0]
    partial = jax.lax.dot_general(
        a, wd, (((1,), (1,)), ((), ())), preferred_element_type=jnp.float32)

    @pl.when(pl.program_id(0) == 0)
    def _():
        out_ref[...] = partial

    @pl.when(pl.program_id(0) > 0)
    def _():
        out_ref[...] += partial


def kernel(hidden_states, gate_weight, e_score_correction_bias, expert_gate,
           expert_up, expert_down, shared_gate, shared_up, shared_down):
    x = hidden_states.reshape(T, H)

    scores = jax.nn.sigmoid(x @ gate_weight.astype(jnp.float32).T)

    w_full = pl.pallas_call(
        _router_kernel,
        out_shape=jax.ShapeDtypeStruct((T, E), jnp.float32),
    )(scores, e_score_correction_bias.reshape(1, E))

    w_ext = jnp.concatenate(
        [w_full, jnp.ones((T, NJ - E), jnp.float32)],
        axis=1).T.reshape(NJ, T, 1)

    wg_total = jnp.concatenate([expert_gate.reshape(E * FFN, H), shared_gate])
    wu_total = jnp.concatenate([expert_up.reshape(E * FFN, H), shared_up])
    wd_total = jnp.concatenate(
        [expert_down,
         shared_down.reshape(H, NJ - E, FFN).transpose(1, 0, 2)], axis=0)

    out = pl.pallas_call(
        _ffn_kernel,
        grid=(NJ,),
        in_specs=[
            pl.BlockSpec((T, H), lambda j: (0, 0)),
            pl.BlockSpec((1, T, 1), lambda j: (j, 0, 0)),
            pl.BlockSpec((BC, H), lambda j: (j, 0)),
            pl.BlockSpec((BC, H), lambda j: (j, 0)),
            pl.BlockSpec((1, H, BC), lambda j: (j, 0, 0)),
        ],
        out_specs=pl.BlockSpec((T, H), lambda j: (0, 0)),
        out_shape=jax.ShapeDtypeStruct((T, H), jnp.float32),
    )(x, w_ext, wg_total, wu_total, wd_total)

    return out.reshape(1, T, H)


# speedup vs baseline: 1.4168x; 1.4168x over previous
import jax
import jax.numpy as jnp
from jax.experimental import pallas as pl

T = 2048
H = 1024
E = 8
FFN = 512
SFFN = 2048
NGROUP = 4
BC = 512
NJS = SFFN // BC
NEG = -1e30


def _router_kernel(scores_ref, bias_ref, w_ref):
    scores = scores_ref[...]
    sfc = scores + bias_ref[...]

    eidx = jax.lax.broadcasted_iota(jnp.int32, (T, E), 1)
    gidx8 = eidx // 2
    gsum_full = jnp.zeros((T, E), jnp.float32)
    for g in range(NGROUP):
        in_g = gidx8 == g
        gsum_g = jnp.sum(jnp.where(in_g, sfc, 0.0), axis=-1, keepdims=True)
        gsum_full = jnp.where(in_g, gsum_g, gsum_full)

    m1 = jnp.max(gsum_full, axis=-1, keepdims=True)
    i1 = jnp.min(jnp.where(gsum_full == m1, gidx8, NGROUP),
                 axis=-1, keepdims=True)
    gs2 = jnp.where(gidx8 == i1, NEG, gsum_full)
    m2 = jnp.max(gs2, axis=-1, keepdims=True)
    i2 = jnp.min(jnp.where(gs2 == m2, gidx8, NGROUP),
                 axis=-1, keepdims=True)
    gsel = (gidx8 == i1) | (gidx8 == i2)
    masked = jnp.where(gsel, sfc, NEG)

    e_m1 = jnp.max(masked, axis=-1, keepdims=True)
    e1 = jnp.min(jnp.where(masked == e_m1, eidx, E), axis=-1, keepdims=True)
    masked2 = jnp.where(eidx == e1, NEG, masked)
    e_m2 = jnp.max(masked2, axis=-1, keepdims=True)
    e2 = jnp.min(jnp.where(masked2 == e_m2, eidx, E), axis=-1, keepdims=True)

    sel1 = (eidx == e1)
    sel2 = (eidx == e2)
    w1 = jnp.sum(jnp.where(sel1, scores, 0.0), axis=-1, keepdims=True)
    w2 = jnp.sum(jnp.where(sel2, scores, 0.0), axis=-1, keepdims=True)
    denom = w1 + w2 + 1e-20
    w_ref[...] = (jnp.where(sel1, w1, 0.0) + jnp.where(sel2, w2, 0.0)) / denom


def _experts_kernel(x_ref, w_ref, wg_ref, wu_ref, wd_ref, out_ref):
    x = x_ref[...]
    wg = wg_ref[0].astype(jnp.bfloat16)
    wu = wu_ref[0].astype(jnp.bfloat16)
    wd = wd_ref[0].astype(jnp.bfloat16)
    g = jax.lax.dot_general(
        x, wg, (((1,), (1,)), ((), ())), preferred_element_type=jnp.float32)
    u = jax.lax.dot_general(
        x, wu, (((1,), (1,)), ((), ())), preferred_element_type=jnp.float32)
    a = (g * jax.nn.sigmoid(g) * u * w_ref[0]).astype(jnp.bfloat16)
    partial = jax.lax.dot_general(
        a, wd, (((1,), (1,)), ((), ())), preferred_element_type=jnp.float32)

    @pl.when(pl.program_id(0) == 0)
    def _():
        out_ref[...] = partial

    @pl.when(pl.program_id(0) > 0)
    def _():
        out_ref[...] += partial


def _shared_kernel(x_ref, routed_ref, wg_ref, wu_ref, wd_ref, out_ref):
    x = x_ref[...]
    wg = wg_ref[...].astype(jnp.bfloat16)
    wu = wu_ref[...].astype(jnp.bfloat16)
    wd = wd_ref[...].astype(jnp.bfloat16)
    g = jax.lax.dot_general(
        x, wg, (((1,), (1,)), ((), ())), preferred_element_type=jnp.float32)
    u = jax.lax.dot_general(
        x, wu, (((1,), (1,)), ((), ())), preferred_element_type=jnp.float32)
    a = (g * jax.nn.sigmoid(g) * u).astype(jnp.bfloat16)
    partial = jax.lax.dot_general(
        a, wd, (((1,), (1,)), ((), ())), preferred_element_type=jnp.float32)

    @pl.when(pl.program_id(0) == 0)
    def _():
        out_ref[...] = routed_ref[...] + partial

    @pl.when(pl.program_id(0) > 0)
    def _():
        out_ref[...] += partial


def kernel(hidden_states, gate_weight, e_score_correction_bias, expert_gate,
           expert_up, expert_down, shared_gate, shared_up, shared_down):
    x = hidden_states.reshape(T, H)
    scores = jax.nn.sigmoid(x @ gate_weight.astype(jnp.float32).T)

    w_full = pl.pallas_call(
        _router_kernel,
        out_shape=jax.ShapeDtypeStruct((T, E), jnp.float32),
    )(scores, e_score_correction_bias.reshape(1, E))
    w3 = w_full.T.reshape(E, T, 1)

    xb = x.astype(jnp.bfloat16)

    routed = pl.pallas_call(
        _experts_kernel,
        grid=(E,),
        in_specs=[
            pl.BlockSpec((T, H), lambda j: (0, 0)),
            pl.BlockSpec((1, T, 1), lambda j: (j, 0, 0)),
            pl.BlockSpec((1, FFN, H), lambda j: (j, 0, 0)),
            pl.BlockSpec((1, FFN, H), lambda j: (j, 0, 0)),
            pl.BlockSpec((1, H, FFN), lambda j: (j, 0, 0)),
        ],
        out_specs=pl.BlockSpec((T, H), lambda j: (0, 0)),
        out_shape=jax.ShapeDtypeStruct((T, H), jnp.float32),
    )(xb, w3, expert_gate, expert_up, expert_down)

    out = pl.pallas_call(
        _shared_kernel,
        grid=(NJS,),
        in_specs=[
            pl.BlockSpec((T, H), lambda j: (0, 0)),
            pl.BlockSpec((T, H), lambda j: (0, 0)),
            pl.BlockSpec((BC, H), lambda j: (j, 0)),
            pl.BlockSpec((BC, H), lambda j: (j, 0)),
            pl.BlockSpec((H, BC), lambda j: (0, j)),
        ],
        out_specs=pl.BlockSpec((T, H), lambda j: (0, 0)),
        out_shape=jax.ShapeDtypeStruct((T, H), jnp.float32),
    )(xb, routed, shared_gate, shared_up, shared_down)

    return out.reshape(1, T, H)
